# baseline (speedup 1.0000x reference)
import functools

import jax
import jax.numpy as jnp
from jax import lax
from jax.experimental import pallas as pl
from jax.experimental.pallas import tpu as pltpu
from jax.experimental.pallas import tpu_sc as plsc

D_E = 64
L_MAX = 2048
V2 = 500000

_cached = None


def _build():
    global _cached
    if _cached is not None:
        return _cached

    info = plsc.get_sparse_core_info()
    NC, NS, L = info.num_cores, info.num_subcores, info.num_lanes
    NW = NC * NS
    BPW = L_MAX // NW
    OROWS = BPW // 2
    NCH = BPW * D_E // L

    mesh = plsc.VectorSubcoreMesh(core_axis_name="c", subcore_axis_name="s")

    @functools.partial(
        pl.kernel,
        mesh=mesh,
        out_type=jax.ShapeDtypeStruct((L_MAX // 2, 128), jnp.float32),
        compiler_params=pltpu.CompilerParams(needs_layout_passes=False),
        scratch_types=[
            pltpu.VMEM((BPW,), jnp.int32),
            pltpu.VMEM((BPW,), jnp.int32),
            pltpu.VMEM((BPW,), jnp.int32),
            pltpu.VMEM((BPW, 128), jnp.float32),
            pltpu.VMEM((OROWS, 128), jnp.float32),
            pltpu.VMEM((OROWS, 128), jnp.float32),
            pltpu.SemaphoreType.DMA,
            pltpu.SemaphoreType.DMA,
        ],
    )
    def emb(x_hbm, tbl_hbm, pos_hbm, out_hbm,
            idx_v, line_v, half_v, rows_v, pos_v, out_v, sem_g, sem_p):
        wid = lax.axis_index("s") * NC + lax.axis_index("c")
        base = wid * BPW
        obase = wid * OROWS

        pltpu.sync_copy(x_hbm.at[pl.ds(base, BPW)], idx_v)
        for i in range(BPW // L):
            sl = pl.ds(i * L, L)
            v = idx_v[sl]
            line_v[sl] = lax.shift_right_logical(v, 1)
            half_v[sl] = lax.bitwise_and(v, 1)

        gather = pltpu.make_async_copy(tbl_hbm.at[line_v], rows_v, sem_g)
        gather.start()
        pos_cp = pltpu.make_async_copy(
            pos_hbm.at[pl.ds(obase, OROWS)], pos_v, sem_p)
        pos_cp.start()
        gather.wait()
        pos_cp.wait()

        iot = lax.iota(jnp.int32, 16)

        def body(c, carry):
            k = c * 16 + iot
            t = lax.shift_right_logical(k, 6)
            d = lax.bitwise_and(k, 63)
            hv = plsc.load_gather(half_v, [t])
            col = d + lax.shift_left(hv, 6)
            gv = plsc.load_gather(rows_v, [t, col])
            r = lax.shift_right_logical(k, 7)
            cc = lax.bitwise_and(k, 127)
            pv = plsc.load_gather(pos_v, [r, cc])
            plsc.store_scatter(out_v, [r, cc], gv + pv)
            return carry

        lax.fori_loop(0, NCH, body, 0)
        pltpu.sync_copy(out_v, out_hbm.at[pl.ds(obase, OROWS)])

    _cached = emb
    return emb


def kernel(x, word_table, pos_table):
    emb = _build()
    tbl = (
        word_table.T.reshape(D_E, V2, 2)
        .transpose(1, 2, 0)
        .reshape(V2, 128)
    )
    pos2 = pos_table.reshape(L_MAX // 2, 128)
    out2 = emb(x.astype(jnp.int32), tbl, pos2)
    return out2.reshape(L_MAX, D_E)


# speedup vs baseline: 1.3964x; 1.3964x over previous
import functools

import jax
import jax.numpy as jnp
from jax import lax
from jax.experimental import pallas as pl
from jax.experimental.pallas import tpu as pltpu
from jax.experimental.pallas import tpu_sc as plsc

D_E = 64
L_MAX = 2048
VOCAB = 1000000

_cached = None


def _build():
    global _cached
    if _cached is not None:
        return _cached

    info = plsc.get_sparse_core_info()
    NC, NS, L = info.num_cores, info.num_subcores, info.num_lanes
    NW = NC * NS
    BPW = L_MAX // NW
    OROWS = BPW // 2
    NCH = BPW * D_E // L

    mesh = plsc.VectorSubcoreMesh(core_axis_name="c", subcore_axis_name="s")

    @functools.partial(
        pl.kernel,
        mesh=mesh,
        out_type=jax.ShapeDtypeStruct((L_MAX // 2, 128), jnp.float32),
        compiler_params=pltpu.CompilerParams(needs_layout_passes=False),
        scratch_types=[
            pltpu.VMEM((BPW,), jnp.int32),
            pltpu.VMEM((BPW, 128), jnp.float32),
            pltpu.VMEM((OROWS, 128), jnp.float32),
            pltpu.VMEM((OROWS, 128), jnp.float32),
            pltpu.SemaphoreType.DMA,
            pltpu.SemaphoreType.DMA,
        ],
    )
    def emb(x_hbm, tbl_hbm, pos_hbm, out_hbm,
            idx_v, rows_v, pos_v, out_v, sem_g, sem_p):
        wid = lax.axis_index("s") * NC + lax.axis_index("c")
        base = wid * BPW
        obase = wid * OROWS

        pltpu.sync_copy(x_hbm.at[pl.ds(base, BPW)], idx_v)
        gather = pltpu.make_async_copy(tbl_hbm.at[idx_v], rows_v, sem_g)
        gather.start()
        pos_cp = pltpu.make_async_copy(
            pos_hbm.at[pl.ds(obase, OROWS)], pos_v, sem_p)
        pos_cp.start()
        gather.wait()
        pos_cp.wait()

        iot = lax.iota(jnp.int32, 16)

        def body(c, carry):
            k = c * 16 + iot
            t = lax.shift_right_logical(k, 6)
            d = lax.bitwise_and(k, 63)
            gv = plsc.load_gather(rows_v, [t, d])
            r = lax.shift_right_logical(k, 7)
            cc = lax.bitwise_and(k, 127)
            pv = plsc.load_gather(pos_v, [r, cc])
            plsc.store_scatter(out_v, [r, cc], gv + pv)
            return carry

        lax.fori_loop(0, NCH, body, 0)
        pltpu.sync_copy(out_v, out_hbm.at[pl.ds(obase, OROWS)])

    _cached = emb
    return emb


def kernel(x, word_table, pos_table):
    emb = _build()
    tblw = jnp.concatenate(
        [word_table, jnp.zeros((VOCAB, 128 - D_E), jnp.float32)], axis=1)
    pos2 = pos_table.reshape(L_MAX // 2, 128)
    out2 = emb(x.astype(jnp.int32), tblw, pos2)
    return out2.reshape(L_MAX, D_E)
